# Initial kernel scaffold; baseline (speedup 1.0000x reference)
import jax
import jax.numpy as jnp
from jax.experimental import pallas as pl


def kernel(x, edge_index, edge_weight, W1, u1, c1, b1, W2, u2, c2, b2, Wl, bl, We1, be1, Wmu, bmu, Wlv, blv, Wd1, bd1, Wd2, bd2):
    raise NotImplementedError("write your pallas kernel here")


# speedup vs baseline: 1.0692x; 1.0692x over previous
import jax
import jax.numpy as jnp
from jax.experimental import pallas as pl

N = 10000
HEADS = 2


def _feast(x, src, dst, W, u, c, b, heads, out_ch):
    xu = x @ u
    q = jax.nn.softmax(xu[src] - xu[dst] + c, axis=-1)
    xw = (x @ W).reshape(x.shape[0], heads, out_ch)
    msg = jnp.einsum('ehc,eh->ec', xw[src], q)
    s = jax.ops.segment_sum(msg, dst, num_segments=N)
    cnt = jax.ops.segment_sum(jnp.ones((dst.shape[0],), jnp.float32), dst, num_segments=N)
    return s / jnp.clip(cnt, 1.0, None)[:, None] + b


def _gcn(x, src, dst, ew, W, b):
    deg = jax.ops.segment_sum(ew, dst, num_segments=N)
    dinv = jnp.where(deg > 0, deg ** -0.5, 0.0)
    norm = dinv[src] * ew * dinv[dst]
    out = jax.ops.segment_sum(norm[:, None] * (x @ W)[src], dst, num_segments=N)
    return out + b


def kernel(x, edge_index, edge_weight, W1, u1, c1, b1, W2, u2, c2, b2, Wl, bl, We1, be1, Wmu, bmu, Wlv, blv, Wd1, bd1, Wd2, bd2):
    loop = jnp.arange(N, dtype=edge_index.dtype)
    src = jnp.concatenate([edge_index[0], loop])
    dst = jnp.concatenate([edge_index[1], loop])
    ew = jnp.concatenate([edge_weight, jnp.ones((N,), jnp.float32)])
    h = jax.nn.relu(_feast(x, src, dst, W1, u1, c1, b1, HEADS, 256))
    h = jax.nn.relu(_feast(h, src, dst, W2, u2, c2, b2, HEADS, 128))
    h = h @ Wl + bl
    he = jax.nn.relu(_gcn(h, src, dst, ew, We1, be1))
    mu = _gcn(he, src, dst, ew, Wmu, bmu)
    logvar = _gcn(he, src, dst, ew, Wlv, blv)
    eps = jax.random.normal(jax.random.key(42), mu.shape, dtype=jnp.float32)
    z = mu + jnp.exp(0.5 * logvar) * eps
    hd = jax.nn.relu(_gcn(z, src, dst, ew, Wd1, bd1))
    recon = _gcn(hd, src, dst, ew, Wd2, bd2)
    return recon, mu, logvar


# speedup vs baseline: 1.4327x; 1.3400x over previous
import jax
import jax.numpy as jnp
from jax import lax
from jax.experimental import pallas as pl
from jax.experimental.pallas import tpu as pltpu
from jax.experimental.pallas import tpu_sc as plsc

N = 10000
NPAD = 10240
NC, NS, L = 2, 16, 16
B = 128
EPAD = 172032
EB = 2048
RB = 1000
HEADS = 2


def _sc_gather(table, idx2, W):
    chunk = EPAD // NS
    nbatch = chunk // B

    def body(t_hbm, idx_hbm, m_hbm, gidx, gbuf0, gbuf1, s_i, s_g, s_o):
        c = lax.axis_index("c")
        s = lax.axis_index("s")
        off = s * chunk
        G = 12
        assert nbatch % G == 0

        def _group(jg, _):
            for t in range(G):
                base = off + jg * (G * B) + t * B
                cpi = pltpu.make_async_copy(idx_hbm.at[c, pl.ds(base, B)],
                                            gidx, s_i)
                cpi.start()
                cpi.wait()
                gbuf = gbuf0 if t % 2 == 0 else gbuf1
                cpg = pltpu.make_async_copy(t_hbm.at[gidx], gbuf, s_g)
                cpg.start()
                cpg.wait()
                cpo = pltpu.make_async_copy(gbuf, m_hbm.at[c, pl.ds(base, B)],
                                            s_o)
                cpo.start()
                cpo.wait()
            return 0
        lax.fori_loop(0, nbatch // G, _group, 0)

    f = pl.kernel(
        body,
        out_type=jax.ShapeDtypeStruct((NC, EPAD, W), jnp.float32),
        mesh=plsc.VectorSubcoreMesh(core_axis_name="c", subcore_axis_name="s"),
        scratch_types=[
            pltpu.VMEM((B,), jnp.int32),
            pltpu.VMEM((B, W), jnp.float32),
            pltpu.VMEM((B, W), jnp.float32),
            pltpu.SemaphoreType.DMA,
            pltpu.SemaphoreType.DMA,
            pltpu.SemaphoreType.DMA,
        ],
    )
    return f(table, idx2)


def _sc_scatter_add(m2, dst2, W):
    EP = m2.shape[1]
    chunk = EP // NS
    nbatch = chunk // B
    zrows = NPAD // NS

    def body(m_hbm, dst_hbm, z_hbm, out_hbm, buf0, buf1, didx, acc,
             s_m, s_i, s_a):
        c = lax.axis_index("c")
        s = lax.axis_index("s")
        cz = pltpu.make_async_copy(z_hbm.at[pl.ds(s * zrows, zrows)],
                                   acc.at[pl.ds(s * zrows, zrows)], s_m)
        cz.start()
        cz.wait()
        plsc.subcore_barrier()
        off = s * chunk
        for j in range(nbatch):
            base = off + j * B
            cpi = pltpu.make_async_copy(dst_hbm.at[c, pl.ds(base, B)], didx, s_i)
            cpi.start()
            buf = buf0 if j % 2 == 0 else buf1
            cpm = pltpu.make_async_copy(m_hbm.at[c, pl.ds(base, B)], buf, s_m)
            cpm.start()
            cpi.wait()
            cpm.wait()
            cpa = pltpu.make_async_copy(buf, acc.at[didx], s_a)
            cpa.start(add=True)
            cpa.wait()
        plsc.subcore_barrier()
        co = pltpu.make_async_copy(acc.at[pl.ds(s * zrows, zrows)],
                                   out_hbm.at[c, pl.ds(s * zrows, zrows)], s_m)
        co.start()
        co.wait()

    f = pl.kernel(
        body,
        out_type=jax.ShapeDtypeStruct((NC, NPAD, W), jnp.float32),
        mesh=plsc.VectorSubcoreMesh(core_axis_name="c", subcore_axis_name="s"),
        scratch_types=[
            pltpu.VMEM((B, W), jnp.float32),
            pltpu.VMEM((B, W), jnp.float32),
            pltpu.VMEM((B,), jnp.int32),
            pltpu.VMEM_SHARED((NPAD, W), jnp.float32),
            pltpu.SemaphoreType.DMA,
            pltpu.SemaphoreType.DMA,
            pltpu.SemaphoreType.DMA,
        ],
    )
    zeros = jnp.zeros((NPAD, W), jnp.float32)
    return f(m2, dst2, zeros)


def _seg_sum(m2, dst2, W):
    s0 = jax.ops.segment_sum(m2[0], dst2[0], num_segments=NPAD)
    s1 = jax.ops.segment_sum(m2[1], dst2[1], num_segments=NPAD)
    return jnp.stack([s0, s1])


def _mm_kernel(x, Ws, outs_w, fuse):
    nb = N // RB

    def body(x_ref, *refs):
        w_refs = refs[:len(Ws)]
        o_refs = refs[len(Ws):]
        res = fuse(x_ref[...], *[w[...] for w in w_refs])
        for o_ref, r in zip(o_refs, res):
            o_ref[...] = r

    in_specs = [pl.BlockSpec((RB, x.shape[1]), lambda i: (i, 0))]
    for w in Ws:
        in_specs.append(pl.BlockSpec(w.shape, lambda i: (0, 0)))
    out_specs = [pl.BlockSpec((RB, w), lambda i: (i, 0)) for w in outs_w]
    out_shape = [jax.ShapeDtypeStruct((N, w), jnp.float32) for w in outs_w]
    f = pl.pallas_call(
        body,
        grid=(nb,),
        in_specs=in_specs,
        out_specs=out_specs if len(outs_w) > 1 else out_specs[0],
        out_shape=out_shape if len(outs_w) > 1 else out_shape[0],
    )
    res = f(x, *Ws)
    return res if isinstance(res, (tuple, list)) else (res,)


def _edge_map(inputs, outs_w, fuse):
    nb = EPAD // EB

    def body(*refs):
        i_refs = refs[:len(inputs)]
        o_refs = refs[len(inputs):]
        vals = []
        for a, r in zip(inputs, i_refs):
            v = r[...]
            if a.ndim == 3:
                v = v[0]
            vals.append(v)
        res = fuse(*vals)
        for o_ref, rr in zip(o_refs, res):
            o_ref[0] = rr

    in_specs = []
    for a in inputs:
        if a.ndim == 3:
            in_specs.append(
                pl.BlockSpec((1, EB, a.shape[2]), lambda k, i: (k, i, 0)))
        elif a.shape == (1, 1):
            in_specs.append(pl.BlockSpec((1, 1), lambda k, i: (0, 0)))
        else:
            in_specs.append(pl.BlockSpec((EB, a.shape[1]), lambda k, i: (i, 0)))
    out_specs = [pl.BlockSpec((1, EB, w), lambda k, i: (k, i, 0))
                 for w in outs_w]
    out_shape = [jax.ShapeDtypeStruct((NC, EPAD, w), jnp.float32)
                 for w in outs_w]
    f = pl.pallas_call(
        body,
        grid=(NC, nb),
        in_specs=in_specs,
        out_specs=out_specs if len(outs_w) > 1 else out_specs[0],
        out_shape=out_shape if len(outs_w) > 1 else out_shape[0],
    )
    res = f(*inputs)
    return res if isinstance(res, (tuple, list)) else (res,)


def _dot(a, b):
    return jax.lax.dot(a, b, preferred_element_type=jnp.float32)


def _planes_from(Y):
    Wc = Y.shape[1] // 2
    return Y.reshape(N, 2, Wc).transpose(1, 0, 2).reshape(2 * N, Wc)


def _concat_planes(S):
    return S[:, :N, :].transpose(1, 0, 2).reshape(N, -1)


def _feast_pass(XW, vtab, cdiff, val_pad, srcg2, dst2w, qidx2, msg_w):
    hc = msg_w // 2
    Vg = _sc_gather(vtab, qidx2, 128)
    T = XW.reshape(N, 2, 2, hc).transpose(2, 0, 1, 3).reshape(2 * N, msg_w)
    M = _sc_gather(T, srcg2, msg_w)
    v_s = Vg[0, :, :1]
    v_d = Vg[1, :, :1]

    def fuse(m, vs, vd, val, cd):
        d10 = vs - vd + cd[0, 0]
        q0 = val / (1.0 + jnp.exp(d10))
        q1 = val - q0
        return (q0 * m[:, :hc] + q1 * m[:, hc:],)

    Mq, = _edge_map([M, v_s, v_d, val_pad[:, None], cdiff], [hc], fuse)
    S = _seg_sum(Mq, dst2w, hc)
    return _concat_planes(S)


def _gcn_pass(Y, ew_pad, srcg2, dst2w):
    Wc = Y.shape[1] // 2
    T = _planes_from(Y)
    M = _sc_gather(T, srcg2, Wc)

    def fuse(m, ew):
        return (ew * m,)

    Ms, = _edge_map([M, ew_pad[:, None]], [Wc], fuse)
    S = _seg_sum(Ms, dst2w, Wc)
    return _concat_planes(S)


def kernel(x, edge_index, edge_weight, W1, u1, c1, b1, W2, u2, c2, b2, Wl, bl,
           We1, be1, Wmu, bmu, Wlv, blv, Wd1, bd1, Wd2, bd2):
    loop = jnp.arange(N, dtype=edge_index.dtype)
    src = jnp.concatenate([edge_index[0], loop])
    dst = jnp.concatenate([edge_index[1], loop])
    ew = jnp.concatenate([edge_weight, jnp.ones((N,), jnp.float32)])
    E0 = src.shape[0]
    pad = EPAD - E0

    src_pad = jnp.concatenate([src, jnp.zeros((pad,), src.dtype)])
    dst_pad = jnp.concatenate([dst, jnp.full((pad,), N, dst.dtype)])
    ew_pad = jnp.concatenate([ew, jnp.zeros((pad,), jnp.float32)])
    val_pad = jnp.concatenate([jnp.ones((E0,), jnp.float32),
                               jnp.zeros((pad,), jnp.float32)])

    srcg2 = jnp.stack([src_pad, src_pad + N])
    dst2w = jnp.stack([dst_pad, dst_pad])
    qidx2 = jnp.stack([src_pad, dst_pad])

    degrows = jnp.pad(jnp.stack([ew_pad, val_pad], axis=1), ((0, 0), (0, 14)))
    D = _seg_sum(degrows.reshape(2, EPAD // 2, 16),
                 dst_pad.reshape(2, EPAD // 2), 16)
    Dsum = D[0, :N] + D[1, :N]
    deg = Dsum[:, 0]
    cnt = Dsum[:, 1]
    dinv = jnp.where(deg > 0, lax.rsqrt(deg), 0.0)[:, None]
    cntinv = (1.0 / jnp.clip(cnt, 1.0, None))[:, None]

    XW1, xu1 = _mm_kernel(x, [W1, u1], [512, 2],
                          lambda xb, w, u: (_dot(xb, w), _dot(xb, u)))
    vt1 = jnp.pad((xu1[:, 1] - xu1[:, 0])[:, None], ((0, 0), (0, 127)))
    cd1 = (c1[1] - c1[0]).reshape(1, 1)
    S1 = _feast_pass(XW1, vt1, cd1, val_pad, srcg2, dst2w, qidx2, 256)
    h1, = _mm_kernel(
        jnp.concatenate([S1, cntinv], axis=1), [b1[None, :]], [256],
        lambda sb, b: (jax.nn.relu(sb[:, :256] * sb[:, 256:257] + b),))

    XW2, xu2 = _mm_kernel(h1, [W2, u2], [256, 2],
                          lambda xb, w, u: (_dot(xb, w), _dot(xb, u)))
    vt2 = jnp.pad((xu2[:, 1] - xu2[:, 0])[:, None], ((0, 0), (0, 127)))
    cd2 = (c2[1] - c2[0]).reshape(1, 1)
    S2 = _feast_pass(XW2, vt2, cd2, val_pad, srcg2, dst2w, qidx2, 128)
    h2, = _mm_kernel(
        jnp.concatenate([S2, cntinv], axis=1), [b2[None, :]], [128],
        lambda sb, b: (jax.nn.relu(sb[:, :128] * sb[:, 128:129] + b),))

    Y1, = _mm_kernel(
        jnp.concatenate([h2, dinv], axis=1), [Wl, bl[None, :], We1], [256],
        lambda hb, wl, bb, we: (
            _dot(_dot(hb[:, :128], wl) + bb, we) * hb[:, 128:129],))
    S3 = _gcn_pass(Y1, ew_pad, srcg2, dst2w)

    Wmulv = jnp.concatenate([Wmu, Wlv], axis=1)
    Y2, = _mm_kernel(
        jnp.concatenate([S3, dinv], axis=1), [be1[None, :], Wmulv], [256],
        lambda sb, b, w: (
            _dot(jax.nn.relu(sb[:, :256] * sb[:, 256:257] + b), w)
            * sb[:, 256:257],))
    S45 = _gcn_pass(Y2, ew_pad, srcg2, dst2w)

    eps = jax.random.normal(jax.random.key(42), (N, 128), dtype=jnp.float32)

    def f_vae(sb, bm, bv, w):
        di = sb[:, 256:257]
        mu_b = sb[:, :128] * di + bm
        lv_b = sb[:, 128:256] * di + bv
        z_b = mu_b + jnp.exp(0.5 * lv_b) * sb[:, 257:]
        return (mu_b, lv_b, _dot(z_b, w) * di)

    mu, logvar, Y5 = _mm_kernel(
        jnp.concatenate([S45, dinv, eps], axis=1),
        [bmu[None, :], blv[None, :], Wd1], [128, 128, 256], f_vae)
    S5 = _gcn_pass(Y5, ew_pad, srcg2, dst2w)

    Y6, = _mm_kernel(
        jnp.concatenate([S5, dinv], axis=1), [bd1[None, :], Wd2], [128],
        lambda sb, b, w: (
            _dot(jax.nn.relu(sb[:, :256] * sb[:, 256:257] + b), w)
            * sb[:, 256:257],))
    S6 = _gcn_pass(jnp.pad(Y6, ((0, 0), (0, 128))), ew_pad, srcg2, dst2w)
    recon, = _mm_kernel(
        jnp.concatenate([S6[:, :128], dinv], axis=1), [bd2[None, :]], [128],
        lambda sb, b: (sb[:, :128] * sb[:, 128:129] + b,))
    return recon, mu, logvar
